# speedup vs baseline: 1.9380x; 1.9380x over previous
import jax
import jax.numpy as jnp
from jax import lax
from jax.experimental import pallas as pl
from jax.experimental.pallas import tpu as pltpu
from jax.experimental.pallas import tpu_sc as plsc

_B, _S, _F = 4, 4096, 2048
_L = 16
_W = 256
_G = _W // _L
_R = 128
_NCHUNK = _S // _R
_TASKS_PER_BATCH = _F // _W


def _cumsum_body(x_hbm, out_hbm, buf):
    core = lax.axis_index("c")
    sub = lax.axis_index("s")
    wid = sub * 2 + core
    b = wid // _TASKS_PER_BATCH
    c0 = (wid % _TASKS_PER_BATCH) * _W

    def chunk_body(k, carries):
        r0 = k * _R
        pltpu.sync_copy(x_hbm.at[b, pl.ds(r0, _R), pl.ds(c0, _W)], buf)

        def row_body(r, cs):
            out = []
            for g in range(_G):
                c = cs[g] + buf[r, pl.ds(g * _L, _L)]
                buf[r, pl.ds(g * _L, _L)] = c
                out.append(c)
            return tuple(out)

        carries = lax.fori_loop(0, _R, row_body, carries)
        pltpu.sync_copy(buf, out_hbm.at[b, pl.ds(r0, _R), pl.ds(c0, _W)])
        return carries

    zeros = tuple(jnp.zeros((_L,), jnp.float32) for _ in range(_G))
    lax.fori_loop(0, _NCHUNK, chunk_body, zeros)


@jax.jit
def _cumsum_axis1(x):
    mesh = plsc.VectorSubcoreMesh(
        core_axis_name="c", subcore_axis_name="s", num_cores=2, num_subcores=16
    )
    return pl.kernel(
        _cumsum_body,
        out_type=jax.ShapeDtypeStruct((_B, _S, _F), jnp.float32),
        mesh=mesh,
        scratch_types=[pltpu.VMEM((_R, _W), jnp.float32)],
    )(x)


def kernel(input_tensor, axis):
    del axis
    return _cumsum_axis1(input_tensor)


# speedup vs baseline: 2.9058x; 1.4994x over previous
import jax
import jax.numpy as jnp
from jax import lax
from jax.experimental import pallas as pl
from jax.experimental.pallas import tpu as pltpu
from jax.experimental.pallas import tpu_sc as plsc

_B, _S, _F = 4, 4096, 2048
_L = 16
_W = 256
_G = _W // _L
_R = 64
_NCHUNK = _S // _R
_TASKS_PER_BATCH = _F // _W


def _cumsum_body(x_hbm, out_hbm, in0, in1, ot0, ot1, si0, si1, so0, so1):
    core = lax.axis_index("c")
    sub = lax.axis_index("s")
    wid = sub * 2 + core
    b = wid // _TASKS_PER_BATCH
    c0 = (wid % _TASKS_PER_BATCH) * _W

    ins, outs, sis, sos = (in0, in1), (ot0, ot1), (si0, si1), (so0, so1)

    def src(k):
        return x_hbm.at[b, pl.ds(k * _R, _R), pl.ds(c0, _W)]

    def dst(k):
        return out_hbm.at[b, pl.ds(k * _R, _R), pl.ds(c0, _W)]

    pltpu.make_async_copy(src(0), in0, si0).start()
    pltpu.make_async_copy(src(1), in1, si1).start()

    def compute(ibuf, obuf, carries):
        def row_body(r, cs):
            res = []
            for g in range(_G):
                c = cs[g] + ibuf[r, pl.ds(g * _L, _L)]
                obuf[r, pl.ds(g * _L, _L)] = c
                res.append(c)
            return tuple(res)

        return lax.fori_loop(0, _R, row_body, carries)

    def pair_body(j, carries):
        for s in range(2):
            k = 2 * j + s
            ibuf, obuf, si, so = ins[s], outs[s], sis[s], sos[s]
            pltpu.make_async_copy(src(k), ibuf, si).wait()

            @pl.when(j > 0)
            def _():
                pltpu.make_async_copy(obuf, dst(k), so).wait()

            carries = compute(ibuf, obuf, carries)
            pltpu.make_async_copy(obuf, dst(k), so).start()

            @pl.when(j < _NCHUNK // 2 - 1)
            def _():
                pltpu.make_async_copy(src(k + 2), ibuf, si).start()

        return carries

    zeros = tuple(jnp.zeros((_L,), jnp.float32) for _ in range(_G))
    lax.fori_loop(0, _NCHUNK // 2, pair_body, zeros)

    pltpu.make_async_copy(ot0, dst(_NCHUNK - 2), so0).wait()
    pltpu.make_async_copy(ot1, dst(_NCHUNK - 1), so1).wait()


@jax.jit
def _cumsum_axis1(x):
    mesh = plsc.VectorSubcoreMesh(
        core_axis_name="c", subcore_axis_name="s", num_cores=2, num_subcores=16
    )
    return pl.kernel(
        _cumsum_body,
        out_type=jax.ShapeDtypeStruct((_B, _S, _F), jnp.float32),
        mesh=mesh,
        scratch_types=[
            pltpu.VMEM((_R, _W), jnp.float32),
            pltpu.VMEM((_R, _W), jnp.float32),
            pltpu.VMEM((_R, _W), jnp.float32),
            pltpu.VMEM((_R, _W), jnp.float32),
            pltpu.SemaphoreType.DMA,
            pltpu.SemaphoreType.DMA,
            pltpu.SemaphoreType.DMA,
            pltpu.SemaphoreType.DMA,
        ],
    )(x)


def kernel(input_tensor, axis):
    del axis
    return _cumsum_axis1(input_tensor)
